# baseline (speedup 1.0000x reference)
import jax
import jax.numpy as jnp
from jax import lax
from jax.experimental import pallas as pl


_B, _C, _H, _W = 8, 1024, 24, 24
_HW = _H * _W


def _argmax_body(x_ref, out_ref):
    x = x_ref[0]
    m = jnp.max(x, axis=0, keepdims=True)
    iota = lax.broadcasted_iota(jnp.int32, (_C, _HW), 0)
    cand = jnp.where(x == m, iota, _C)
    out_ref[0, 0] = jnp.min(cand, axis=0).astype(jnp.int32)


def kernel(inputs):
    x3 = inputs.reshape(_B, _C, _HW)
    idx = pl.pallas_call(
        _argmax_body,
        grid=(_B,),
        in_specs=[pl.BlockSpec((1, _C, _HW), lambda b: (b, 0, 0))],
        out_specs=pl.BlockSpec((1, 1, _HW), lambda b: (b, 0, 0)),
        out_shape=jax.ShapeDtypeStruct((_B, 1, _HW), jnp.int32),
    )(x3)
    return (inputs, idx.reshape(_B, 1, _H, _W))
